# Initial kernel scaffold; baseline (speedup 1.0000x reference)
import jax
import jax.numpy as jnp
from jax.experimental import pallas as pl


def kernel(z, x, z_emb, W, b):
    raise NotImplementedError("write your pallas kernel here")


# speedup vs baseline: 1.0221x; 1.0221x over previous
import functools

import jax
import jax.numpy as jnp
from jax import lax
from jax.experimental import pallas as pl
from jax.experimental.pallas import tpu as pltpu
from jax.experimental.pallas import tpu_sc as plsc

N = 100000
H = 128
XD = 11
T = 10
L = 16

NC = 2
NS = 16
NW = NC * NS

C = 400
NCH = N // C
CPW = (NCH + NW - 1) // NW
RG = C // L
GROUPS = H // L
RB = 4


def _bcast(vec, lane):
    idx = jnp.full((L, 1), lane, dtype=jnp.int32)
    dnums = lax.GatherDimensionNumbers(
        offset_dims=(), collapsed_slice_dims=(0,), start_index_map=(0,))
    return lax.gather(vec, idx, dnums, (1,),
                      mode=lax.GatherScatterMode.PROMISE_IN_BOUNDS)


def _body(z_h, xf_h, emb_h, wt_h, b_h, out_h, zv, xv, outv, tblv, wtv, bv):
    cid = lax.axis_index("c")
    sid = lax.axis_index("s")
    w = sid * NC + cid

    pltpu.sync_copy(emb_h, tblv)
    pltpu.sync_copy(wt_h, wtv)
    pltpu.sync_copy(b_h, bv)
    for t in range(T):
        for g in range(GROUPS):
            sl = pl.ds(t * H + g * L, L)
            tblv[sl] = tblv[sl] + bv[pl.ds(g * L, L)]

    iota = lax.iota(jnp.int32, L)

    def chunk_body(i, carry):
        c = w + NW * i

        @pl.when(c < NCH)
        def _():
            base = c * C
            pltpu.sync_copy(z_h.at[pl.ds(base, C)], zv.at[pl.ds(0, C)])
            pltpu.sync_copy(xf_h.at[pl.ds(base * XD, C * XD)],
                            xv.at[pl.ds(0, C * XD)])

            def rowgrp(g16, _):
                zv16 = zv[pl.ds(g16 * L, L)]
                for blk in range(L // RB):
                    rows = [g16 * L + blk * RB + t for t in range(RB)]
                    xr = [xv[pl.ds(r * XD, L)] for r in rows]
                    zb = [_bcast(zv16, blk * RB + t) * H + iota
                          for t in range(RB)]
                    acc = [[plsc.load_gather(tblv, [zb[t] + g * L])
                            for g in range(GROUPS)] for t in range(RB)]
                    for k in range(XD):
                        wk = [wtv[k, pl.ds(g * L, L)] for g in range(GROUPS)]
                        xb = [_bcast(xr[t], k) for t in range(RB)]
                        for t in range(RB):
                            for g in range(GROUPS):
                                acc[t][g] = acc[t][g] + xb[t] * wk[g]
                    for t in range(RB):
                        for g in range(GROUPS):
                            outv[rows[t], pl.ds(g * L, L)] = acc[t][g]
                return 0

            lax.fori_loop(0, RG, rowgrp, 0)

            pltpu.sync_copy(outv, out_h.at[pl.ds(base, C)])

        return carry

    lax.fori_loop(0, CPW, chunk_body, 0)


@jax.jit
def _sc_encode(z, xf, z_emb, wt, b):
    mesh = plsc.VectorSubcoreMesh(
        core_axis_name="c", subcore_axis_name="s",
        num_cores=NC, num_subcores=NS,
    )
    return pl.kernel(
        _body,
        out_type=jax.ShapeDtypeStruct((N, H), jnp.float32),
        mesh=mesh,
        compiler_params=pltpu.CompilerParams(needs_layout_passes=False),
        scratch_types=[
            pltpu.VMEM((C,), jnp.int32),
            pltpu.VMEM((C * XD + L,), jnp.float32),
            pltpu.VMEM((C, H), jnp.float32),
            pltpu.VMEM((T * H,), jnp.float32),
            pltpu.VMEM((XD, H), jnp.float32),
            pltpu.VMEM((H,), jnp.float32),
        ],
    )(z, xf, z_emb, wt, b)


def kernel(z, x, z_emb, W, b):
    z = z.astype(jnp.int32)
    xf = x.reshape(-1)
    wt = W.T
    return _sc_encode(z, xf, z_emb.reshape(-1), wt, b)


# speedup vs baseline: 1.0883x; 1.0648x over previous
import jax
import jax.numpy as jnp
from jax import lax
from jax.experimental import pallas as pl
from jax.experimental.pallas import tpu as pltpu
from jax.experimental.pallas import tpu_sc as plsc

N = 100000
H = 128
XD = 11
T = 10
L = 16

NC = 2
NS = 16
NW = NC * NS

C = 400
NCH = N // C
CPW = (NCH + NW - 1) // NW
RG = C // L
GROUPS = H // L
RB = 4


def _bcast(vec, lane):
    idx = jnp.full((L, 1), lane, dtype=jnp.int32)
    dnums = lax.GatherDimensionNumbers(
        offset_dims=(), collapsed_slice_dims=(0,), start_index_map=(0,))
    return lax.gather(vec, idx, dnums, (1,),
                      mode=lax.GatherScatterMode.PROMISE_IN_BOUNDS)


def _body(z_h, xf_h, emb_h, wt_h, b_h, out_h,
          zv0, xv0, ov0, zv1, xv1, ov1, tblv, wtv, bv,
          si0, si1, so0, so1):
    cid = lax.axis_index("c")
    sid = lax.axis_index("s")
    w = sid * NC + cid

    pltpu.sync_copy(emb_h, tblv)
    pltpu.sync_copy(wt_h, wtv)
    pltpu.sync_copy(b_h, bv)
    for t in range(T):
        for g in range(GROUPS):
            sl = pl.ds(t * H + g * L, L)
            tblv[sl] = tblv[sl] + bv[pl.ds(g * L, L)]

    iota = lax.iota(jnp.int32, L)
    bufs = ((zv0, xv0, ov0, si0, so0), (zv1, xv1, ov1, si1, so1))

    def in_pair(ci, zb, xb, sem):
        zcp = pltpu.make_async_copy(
            z_h.at[pl.ds(ci * C, C)], zb.at[pl.ds(0, C)], sem)
        xcp = pltpu.make_async_copy(
            xf_h.at[pl.ds(ci * C * XD, C * XD)], xb.at[pl.ds(0, C * XD)], sem)
        return zcp, xcp

    def start_in(ci, zb, xb, sem):
        @pl.when(ci < NCH)
        def _():
            zcp, xcp = in_pair(ci, zb, xb, sem)
            zcp.start()
            xcp.start()

    def wait_in(ci, zb, xb, sem):
        @pl.when(ci < NCH)
        def _():
            zcp, xcp = in_pair(ci, zb, xb, sem)
            zcp.wait()
            xcp.wait()

    def out_desc(ci, ob, sem):
        return pltpu.make_async_copy(ob, out_h.at[pl.ds(ci * C, C)], sem)

    def start_out(ci, ob, sem):
        @pl.when(ci < NCH)
        def _():
            out_desc(ci, ob, sem).start()

    def wait_out(ci, ob, sem):
        @pl.when((ci >= 0) & (ci < NCH))
        def _():
            out_desc(ci, ob, sem).wait()

    def compute(zv, xv, outv):
        def rowgrp(g16, carry):
            zv16 = zv[pl.ds(g16 * L, L)]
            for blk in range(L // RB):
                rows = [g16 * L + blk * RB + t for t in range(RB)]
                xr = [xv[pl.ds(r * XD, L)] for r in rows]
                zb = [_bcast(zv16, blk * RB + t) * H + iota
                      for t in range(RB)]
                acc = [[plsc.load_gather(tblv, [zb[t] + g * L])
                        for g in range(GROUPS)] for t in range(RB)]
                for k in range(XD):
                    wk = [wtv[k, pl.ds(g * L, L)] for g in range(GROUPS)]
                    xb = [_bcast(xr[t], k) for t in range(RB)]
                    for t in range(RB):
                        for g in range(GROUPS):
                            acc[t][g] = acc[t][g] + xb[t] * wk[g]
                for t in range(RB):
                    for g in range(GROUPS):
                        outv[rows[t], pl.ds(g * L, L)] = acc[t][g]
            return carry

        lax.fori_loop(0, RG, rowgrp, 0)

    start_in(w, zv0, xv0, si0)

    def loop_body(ii, carry):
        for p in range(2):
            zv, xv, ov, si, so = bufs[p]
            nzv, nxv, _, nsi, _ = bufs[1 - p]
            c = w + NW * (2 * ii + p)
            start_in(c + NW, nzv, nxv, nsi)
            wait_in(c, zv, xv, si)
            wait_out(c - 2 * NW, ov, so)

            @pl.when(c < NCH)
            def _():
                compute(zv, xv, ov)

            start_out(c, ov, so)
        return carry

    lax.fori_loop(0, CPW // 2, loop_body, 0)

    wait_out(w + NW * (CPW - 2), ov0, so0)
    wait_out(w + NW * (CPW - 1), ov1, so1)


@jax.jit
def _sc_encode(z, xf, z_emb, wt, b):
    mesh = plsc.VectorSubcoreMesh(
        core_axis_name="c", subcore_axis_name="s",
        num_cores=NC, num_subcores=NS,
    )
    return pl.kernel(
        _body,
        out_type=jax.ShapeDtypeStruct((N, H), jnp.float32),
        mesh=mesh,
        compiler_params=pltpu.CompilerParams(needs_layout_passes=False),
        scratch_types=[
            pltpu.VMEM((C,), jnp.int32),
            pltpu.VMEM((C * XD + L,), jnp.float32),
            pltpu.VMEM((C, H), jnp.float32),
            pltpu.VMEM((C,), jnp.int32),
            pltpu.VMEM((C * XD + L,), jnp.float32),
            pltpu.VMEM((C, H), jnp.float32),
            pltpu.VMEM((T * H,), jnp.float32),
            pltpu.VMEM((XD, H), jnp.float32),
            pltpu.VMEM((H,), jnp.float32),
            pltpu.SemaphoreType.DMA,
            pltpu.SemaphoreType.DMA,
            pltpu.SemaphoreType.DMA,
            pltpu.SemaphoreType.DMA,
        ],
    )(z, xf, z_emb, wt, b)


def kernel(z, x, z_emb, W, b):
    z = z.astype(jnp.int32)
    xf = x.reshape(-1)
    wt = W.T
    return _sc_encode(z, xf, z_emb.reshape(-1), wt, b)
